# Initial kernel scaffold; baseline (speedup 1.0000x reference)
import jax
import jax.numpy as jnp
from jax.experimental import pallas as pl


def kernel(A_from, A_to, W, b):
    raise NotImplementedError("write your pallas kernel here")


# speedup vs baseline: 14.4922x; 14.4922x over previous
import jax
import jax.numpy as jnp
from jax import lax
from jax.experimental import pallas as pl
from jax.experimental.pallas import tpu as pltpu


def _logaddexp(a, b):
    mx = jnp.maximum(a, b)
    d = jnp.abs(a - b)
    return mx + jnp.log1p(jnp.exp(-d))


def _fused_kernel(x1_ref, x2_ref, w_ref, bt_ref, scores_ref, *out_refs):
    x1 = x1_ref[...]
    x2 = x2_ref[...]
    a_dim = x1.shape[1]
    w1 = w_ref[:a_dim, :]
    w2 = w_ref[a_dim:, :]

    st = lax.dot_general(w1, x1, (((0,), (1,)), ((), ())),
                         preferred_element_type=jnp.float32)
    st = st + lax.dot_general(w2, x2, (((0,), (0,)), ((), ())),
                              preferred_element_type=jnp.float32)
    st = st + bt_ref[...]

    scores_ref[...] = jnp.transpose(st)

    ll, tk = st.shape
    clse = st
    shift = 1
    neg_inf = jnp.float32(-jnp.inf)
    while shift < ll:
        shifted = jnp.concatenate(
            [jnp.full((shift, tk), neg_inf, jnp.float32), clse[:-shift, :]],
            axis=0,
        )
        clse = _logaddexp(clse, shifted)
        shift *= 2

    for idx, l in enumerate(range(2, ll + 1)):
        out_refs[idx][...] = st[:l, :] - clse[l - 1:l, :]


def _pick_tile(k_total):
    for t in (512, 256, 128, 64, 32, 16, 8):
        if k_total % t == 0:
            return t
    return k_total


def kernel(A_from, A_to, W, b):
    k_total, a_dim = A_from.shape
    l_dim = W.shape[1]
    tile_k = _pick_tile(k_total)
    grid = (k_total // tile_k,)

    bt = jnp.reshape(b.astype(jnp.float32), (l_dim, 1))

    out_shape = [jax.ShapeDtypeStruct((k_total, l_dim), jnp.float32)]
    out_specs = [pl.BlockSpec((tile_k, l_dim), lambda i: (i, 0))]
    for l in range(2, l_dim + 1):
        out_shape.append(jax.ShapeDtypeStruct((l, k_total), jnp.float32))
        out_specs.append(pl.BlockSpec((l, tile_k), lambda i: (0, i)))

    scores, *lps = pl.pallas_call(
        _fused_kernel,
        grid=grid,
        out_shape=tuple(out_shape),
        in_specs=[
            pl.BlockSpec((tile_k, a_dim), lambda i: (i, 0)),
            pl.BlockSpec((a_dim, tile_k), lambda i: (0, i)),
            pl.BlockSpec((2 * a_dim, l_dim), lambda i: (0, 0)),
            pl.BlockSpec((l_dim, 1), lambda i: (0, 0)),
        ],
        out_specs=tuple(out_specs),
        compiler_params=pltpu.CompilerParams(
            dimension_semantics=("parallel",),
        ),
    )(A_from, A_to, W, bt)

    lplist = [jnp.zeros((1, k_total), jnp.float32)] + lps
    return lplist, scores


# speedup vs baseline: 16.0831x; 1.1098x over previous
import jax
import jax.numpy as jnp
from jax import lax
from jax.experimental import pallas as pl
from jax.experimental.pallas import tpu as pltpu


def _fused_kernel(x1_ref, x2_ref, w_ref, bt_ref, scores_ref, *out_refs):
    x1 = x1_ref[...]
    x2 = x2_ref[...]
    a_dim = x1.shape[1]
    w1 = w_ref[:a_dim, :]
    w2 = w_ref[a_dim:, :]

    st = lax.dot_general(w1, x1, (((0,), (1,)), ((), ())),
                         preferred_element_type=jnp.float32)
    st = st + lax.dot_general(w2, x2, (((0,), (0,)), ((), ())),
                              preferred_element_type=jnp.float32)
    st = st + bt_ref[...]

    scores_ref[...] = jnp.transpose(st)

    ll, tk = st.shape
    m_row = jnp.max(st, axis=0, keepdims=True)
    cs = jnp.exp(st - m_row)
    shift = 1
    while shift < ll:
        shifted = jnp.concatenate(
            [jnp.zeros((shift, tk), jnp.float32), cs[:-shift, :]], axis=0
        )
        cs = cs + shifted
        shift *= 2
    clse = m_row + jnp.log(jnp.maximum(cs, jnp.float32(1e-37)))

    for idx, l in enumerate(range(2, ll + 1)):
        out_refs[idx][...] = st[:l, :] - clse[l - 1:l, :]


def _pick_tile(k_total):
    for t in (1024, 512, 256, 128, 64, 32, 16, 8):
        if k_total % t == 0:
            return t
    return k_total


def kernel(A_from, A_to, W, b):
    k_total, a_dim = A_from.shape
    l_dim = W.shape[1]
    tile_k = _pick_tile(k_total)
    grid = (k_total // tile_k,)

    bt = jnp.reshape(b.astype(jnp.float32), (l_dim, 1))

    out_shape = [jax.ShapeDtypeStruct((k_total, l_dim), jnp.float32)]
    out_specs = [pl.BlockSpec((tile_k, l_dim), lambda i: (i, 0))]
    for l in range(2, l_dim + 1):
        out_shape.append(jax.ShapeDtypeStruct((l, k_total), jnp.float32))
        out_specs.append(pl.BlockSpec((l, tile_k), lambda i: (0, i)))

    scores, *lps = pl.pallas_call(
        _fused_kernel,
        grid=grid,
        out_shape=tuple(out_shape),
        in_specs=[
            pl.BlockSpec((tile_k, a_dim), lambda i: (i, 0)),
            pl.BlockSpec((a_dim, tile_k), lambda i: (0, i)),
            pl.BlockSpec((2 * a_dim, l_dim), lambda i: (0, 0)),
            pl.BlockSpec((l_dim, 1), lambda i: (0, 0)),
        ],
        out_specs=tuple(out_specs),
        compiler_params=pltpu.CompilerParams(
            dimension_semantics=("parallel",),
        ),
    )(A_from, A_to, W, bt)

    lplist = [jnp.zeros((1, k_total), jnp.float32)] + lps
    return lplist, scores
